# baseline (speedup 1.0000x reference)
import functools

import jax
import jax.numpy as jnp
from jax.experimental import pallas as pl
from jax.experimental.pallas import tpu as pltpu
from jax.experimental.pallas import tpu_sc as plsc

EMB = 32
N_FIELDS = 4
ST = 8
BT = 128
GB = 8


def kernel(x, T0, T1, T2, T3):
    F, B, S = x.shape
    x = x.astype(jnp.int32)

    x5 = (
        x.transpose(0, 2, 1)
        .reshape(F, S // ST, ST, B // BT, BT)
        .transpose(0, 1, 3, 2, 4)
    )

    mesh = plsc.VectorSubcoreMesh(
        core_axis_name="core", subcore_axis_name="subcore"
    )

    @functools.partial(
        pl.kernel,
        out_type=jax.ShapeDtypeStruct((B, S, N_FIELDS * EMB), jnp.float32),
        mesh=mesh,
        compiler_params=pltpu.CompilerParams(use_tc_tiling_on_sc=False),
    )
    def sc_gather(x_hbm, t0, t1, t2, t3, out_hbm):
        tabs = [t0, t1, t2, t3]
        for f in range(N_FIELDS):
            table = tabs[f]

            def body(i_vmem, o_vmem, table=table):
                for k in range(GB):
                    pltpu.sync_copy(
                        table.at[i_vmem.at[0, 0, k, 0]],
                        o_vmem.at[pl.ds(k * BT, BT), 0],
                    )

            pltpu.emit_pipeline(
                body,
                grid=(S // ST, B // (GB * BT), ST),
                in_specs=[
                    pl.BlockSpec(
                        (1, 1, GB, 1, BT),
                        index_map=lambda st, g, r, f=f: (f, st, g, r, 0),
                    )
                ],
                out_specs=[
                    pl.BlockSpec(
                        (GB * BT, 1, EMB),
                        index_map=lambda st, g, r, f=f: (g, st * ST + r, f),
                    )
                ],
                core_axis_name=("core", "subcore"),
                dimension_semantics=(
                    pltpu.PARALLEL,
                    pltpu.PARALLEL,
                    pltpu.PARALLEL,
                ),
            )(x_hbm, out_hbm)

    out = sc_gather(x5, T0, T1, T2, T3)
    return out


# speedup vs baseline: 2.0185x; 2.0185x over previous
import functools

import jax
import jax.numpy as jnp
import numpy as np
from jax.experimental import pallas as pl
from jax.experimental.pallas import tpu as pltpu
from jax.experimental.pallas import tpu_sc as plsc

EMB = 32
N_FIELDS = 4
ST = 8
BT = 128
TILE = ST * BT


def kernel(x, T0, T1, T2, T3):
    F, B, S = x.shape
    x = x.astype(jnp.int32)

    n_st = S // ST
    n_bt = B // BT
    steps = n_st * n_bt

    x4 = (
        x.transpose(0, 2, 1)
        .reshape(F, n_st, ST, n_bt, BT)
        .transpose(0, 1, 3, 2, 4)
        .reshape(F, n_st, n_bt, TILE)
    )

    si = np.arange(ST, dtype=np.int32)[:, None]
    bi = np.arange(BT, dtype=np.int32)[None, :]
    pat = bi * (S * N_FIELDS) + si * N_FIELDS
    pats = jnp.asarray(
        np.stack([pat + f for f in range(N_FIELDS)]).reshape(N_FIELDS, TILE)
    )
    span = (BT - 1) * S * N_FIELDS + (ST - 1) * N_FIELDS + N_FIELDS

    mesh = plsc.VectorSubcoreMesh(
        core_axis_name="core", subcore_axis_name="subcore"
    )

    @functools.partial(
        pl.kernel,
        out_type=jax.ShapeDtypeStruct((B * S * N_FIELDS, EMB), jnp.float32),
        mesh=mesh,
        compiler_params=pltpu.CompilerParams(use_tc_tiling_on_sc=False),
        scratch_types=[pltpu.VMEM((TILE, EMB), jnp.float32)],
    )
    def sc_gather(x_hbm, p_hbm, t0, t1, t2, t3, out_hbm, rows):
        tabs = [t0, t1, t2, t3]
        for f in range(N_FIELDS):
            table = tabs[f]

            def body(i_vmem, p_vmem, table=table):
                i = pl.program_id(0)
                st = i // n_bt
                bt = i % n_bt
                base = bt * (BT * S * N_FIELDS) + st * (ST * N_FIELDS)
                pltpu.sync_copy(table.at[i_vmem], rows)
                pltpu.sync_copy(
                    rows, out_hbm.at[pl.ds(base, span)].at[p_vmem.at[0]]
                )

            pltpu.emit_pipeline(
                body,
                grid=(steps,),
                in_specs=[
                    pl.BlockSpec(
                        (None, None, None, TILE),
                        index_map=lambda i, f=f: (f, i // n_bt, i % n_bt, 0),
                    ),
                    pl.BlockSpec((1, TILE), index_map=lambda i, f=f: (f, 0)),
                ],
                core_axis_name=("core", "subcore"),
                dimension_semantics=(pltpu.PARALLEL,),
            )(x_hbm, p_hbm)

    out = sc_gather(x4, pats, T0, T1, T2, T3)
    return out.reshape(B, S, N_FIELDS * EMB)


# speedup vs baseline: 2.2384x; 1.1090x over previous
import functools

import jax
import jax.numpy as jnp
from jax.experimental import pallas as pl
from jax.experimental.pallas import tpu as pltpu
from jax.experimental.pallas import tpu_sc as plsc

EMB = 32
N_FIELDS = 4
WIN = 1024


def kernel(x, T0, T1, T2, T3):
    F, B, S = x.shape
    x = x.astype(jnp.int32)

    xt = jnp.transpose(x, (0, 2, 1))

    mesh = plsc.VectorSubcoreMesh(
        core_axis_name="core", subcore_axis_name="subcore"
    )

    @functools.partial(
        pl.kernel,
        out_type=jax.ShapeDtypeStruct((B, S, N_FIELDS * EMB), jnp.float32),
        mesh=mesh,
        compiler_params=pltpu.CompilerParams(use_tc_tiling_on_sc=False),
    )
    def sc_gather(x_hbm, t0, t1, t2, t3, out_hbm):
        tabs = [t0, t1, t2, t3]
        for f in range(N_FIELDS):
            table = tabs[f]

            def body(i_vmem, o_vmem, table=table):
                pltpu.sync_copy(table.at[i_vmem.at[0, 0]], o_vmem.at[:, 0])

            nj = B // WIN
            pltpu.emit_pipeline(
                body,
                grid=(S * nj,),
                in_specs=[
                    pl.BlockSpec(
                        (1, 1, WIN),
                        index_map=lambda i, f=f: (f, i // nj, i % nj),
                    )
                ],
                out_specs=[
                    pl.BlockSpec(
                        (WIN, 1, EMB),
                        index_map=lambda i, f=f: (i % nj, i // nj, f),
                    )
                ],
                core_axis_name=("core", "subcore"),
                dimension_semantics=(pltpu.PARALLEL,),
            )(x_hbm, out_hbm)

    out = sc_gather(xt, T0, T1, T2, T3)
    return out
